# Initial kernel scaffold; baseline (speedup 1.0000x reference)
import jax
import jax.numpy as jnp
from jax.experimental import pallas as pl


def kernel(feature, edge_index, review_feat, ci, W):
    raise NotImplementedError("write your pallas kernel here")


# speedup vs baseline: 3.8097x; 3.8097x over previous
import functools

import jax
import jax.numpy as jnp
from jax import lax
from jax.experimental import pallas as pl
from jax.experimental.pallas import tpu as pltpu
from jax.experimental.pallas import tpu_sc as plsc

N_NODES = 50000
N_EDGES = 800000
D = 64
DH = 32
L = 16
NC, NS = 2, 16
CHUNK = 128
N_CHUNKS = N_EDGES // CHUNK
ROW_CHUNK = 128
N_ROW_CHUNKS = N_NODES // ROW_CHUNK
ROW_TAIL = N_NODES - N_ROW_CHUNKS * ROW_CHUNK
MM_BLK = 4000


def _matmul_body(rf_ref, wt_ref, r0_ref, r1_ref):
    y = jnp.dot(rf_ref[...], wt_ref[...], preferred_element_type=jnp.float32)
    r0_ref[...] = y[:, :DH]
    r1_ref[...] = y[:, DH:]


def _edge_linear(review_feat, w_t):
    return pl.pallas_call(
        _matmul_body,
        grid=(N_EDGES // MM_BLK,),
        in_specs=[
            pl.BlockSpec((MM_BLK, D), lambda i: (i, 0)),
            pl.BlockSpec((D, D), lambda i: (0, 0)),
        ],
        out_specs=[
            pl.BlockSpec((MM_BLK, DH), lambda i: (i, 0)),
            pl.BlockSpec((MM_BLK, DH), lambda i: (i, 0)),
        ],
        out_shape=[
            jax.ShapeDtypeStruct((N_EDGES, DH), jnp.float32),
            jax.ShapeDtypeStruct((N_EDGES, DH), jnp.float32),
        ],
    )(review_feat, w_t)


def _sc_body(r_hbm, f_hbm, src_hbm, dst_hbm, ci_hbm, out_hbm,
             h_sh, cig_v, cin_v, src_v, dst_v, r_v, f_v, m_v, d_v, sem, sem2, s):
    def _zero_row(i, _):
        zero = jnp.zeros((L,), jnp.float32)
        d_v[i, pl.ds(0, L)] = zero
        d_v[i, pl.ds(L, L)] = zero
        return 0

    lax.fori_loop(0, ROW_CHUNK, _zero_row, 0)

    n_rc = N_ROW_CHUNKS // NS + jnp.where(s < N_ROW_CHUNKS % NS, 1, 0)

    def _zero_chunk(k, _):
        base = (s + NS * k) * ROW_CHUNK
        pltpu.sync_copy(d_v, h_sh.at[pl.ds(base, ROW_CHUNK)])
        return 0

    lax.fori_loop(0, n_rc, _zero_chunk, 0)

    @pl.when(s == NS - 1)
    def _():
        pltpu.sync_copy(d_v.at[pl.ds(0, ROW_TAIL)],
                        h_sh.at[pl.ds(N_ROW_CHUNKS * ROW_CHUNK, ROW_TAIL)])

    plsc.subcore_barrier()

    n_my = N_CHUNKS // NS + jnp.where(s < N_CHUNKS % NS, 1, 0)

    def _chunk_body(i, _):
        e0 = (s + NS * i) * CHUNK
        pltpu.sync_copy(src_hbm.at[pl.ds(e0, CHUNK)], src_v)
        pltpu.sync_copy(dst_hbm.at[pl.ds(e0, CHUNK)], dst_v)
        pltpu.sync_copy(r_hbm.at[pl.ds(e0, CHUNK)], r_v)
        fcp = pltpu.async_copy(f_hbm.at[src_v], f_v, sem)
        ccp = pltpu.async_copy(ci_hbm.at[src_v], cig_v, sem2)
        fcp.wait()
        ccp.wait()

        def _group(g, _):
            base = g * L
            civ = cig_v[pl.ds(base, L)]
            for j in range(L):
                e = base + j
                cse = civ[j]
                m_v[e, pl.ds(0, L)] = (r_v[e, pl.ds(0, L)] + f_v[e, pl.ds(0, L)]) * cse
                m_v[e, pl.ds(L, L)] = (r_v[e, pl.ds(L, L)] + f_v[e, pl.ds(L, L)]) * cse
            return 0

        lax.fori_loop(0, CHUNK // L, _group, 0)
        pltpu.sync_copy(m_v, h_sh.at[dst_v], add=True)
        return 0

    lax.fori_loop(0, n_my, _chunk_body, 0)
    plsc.subcore_barrier()

    def _scale_rows(ngroups):
        def _rgroup(g, _):
            rb = g * L
            civ = cin_v[pl.ds(rb, L)]
            for j in range(L):
                cn = civ[j]
                d_v[rb + j, pl.ds(0, L)] = d_v[rb + j, pl.ds(0, L)] * cn
                d_v[rb + j, pl.ds(L, L)] = d_v[rb + j, pl.ds(L, L)] * cn
            return 0

        lax.fori_loop(0, ngroups, _rgroup, 0)

    def _drain_chunk(k, _):
        base = (s + NS * k) * ROW_CHUNK
        pltpu.sync_copy(h_sh.at[pl.ds(base, ROW_CHUNK)], d_v)
        pltpu.sync_copy(ci_hbm.at[pl.ds(base, ROW_CHUNK)], cin_v)
        _scale_rows(ROW_CHUNK // L)
        pltpu.sync_copy(d_v, out_hbm.at[pl.ds(base, ROW_CHUNK)])
        return 0

    lax.fori_loop(0, n_rc, _drain_chunk, 0)

    @pl.when(s == NS - 1)
    def _():
        base = N_ROW_CHUNKS * ROW_CHUNK
        pltpu.sync_copy(h_sh.at[pl.ds(base, ROW_TAIL)], d_v.at[pl.ds(0, ROW_TAIL)])
        pltpu.sync_copy(ci_hbm.at[pl.ds(base, ROW_TAIL)], cin_v.at[pl.ds(0, ROW_TAIL)])
        _scale_rows(ROW_TAIL // L)
        pltpu.sync_copy(d_v.at[pl.ds(0, ROW_TAIL)], out_hbm.at[pl.ds(base, ROW_TAIL)])


def _sc_kernel(r0, r1, f0, f1, src_hbm, dst_hbm, ci_hbm, out0, out1,
               h_sh, cig_v, cin_v, src_v, dst_v, r_v, f_v, m_v, d_v, sem, sem2):
    c = lax.axis_index("c")
    s = lax.axis_index("s")

    @pl.when(c == 0)
    def _():
        _sc_body(r0, f0, src_hbm, dst_hbm, ci_hbm, out0,
                 h_sh, cig_v, cin_v, src_v, dst_v, r_v, f_v, m_v, d_v,
                 sem, sem2, s)

    @pl.when(c == 1)
    def _():
        _sc_body(r1, f1, src_hbm, dst_hbm, ci_hbm, out1,
                 h_sh, cig_v, cin_v, src_v, dst_v, r_v, f_v, m_v, d_v,
                 sem, sem2, s)


_sc_call = functools.partial(
    pl.kernel,
    _sc_kernel,
    out_type=[
        jax.ShapeDtypeStruct((N_NODES, DH), jnp.float32),
        jax.ShapeDtypeStruct((N_NODES, DH), jnp.float32),
    ],
    mesh=plsc.VectorSubcoreMesh(
        core_axis_name="c", subcore_axis_name="s",
        num_cores=NC, num_subcores=NS),
    scratch_types=[
        pltpu.VMEM_SHARED((N_NODES, DH), jnp.float32),
        pltpu.VMEM((CHUNK,), jnp.float32),
        pltpu.VMEM((ROW_CHUNK,), jnp.float32),
        pltpu.VMEM((CHUNK,), jnp.int32),
        pltpu.VMEM((CHUNK,), jnp.int32),
        pltpu.VMEM((CHUNK, DH), jnp.float32),
        pltpu.VMEM((CHUNK, DH), jnp.float32),
        pltpu.VMEM((CHUNK, DH), jnp.float32),
        pltpu.VMEM((ROW_CHUNK, DH), jnp.float32),
        pltpu.SemaphoreType.DMA,
        pltpu.SemaphoreType.DMA,
    ],
    compiler_params=pltpu.CompilerParams(use_tc_tiling_on_sc=False),
)()


@jax.jit
def kernel(feature, edge_index, review_feat, ci, W):
    src = edge_index[0].astype(jnp.int32)
    dst = edge_index[1].astype(jnp.int32)
    ci_flat = ci[:, 0]
    f0 = feature[:, :DH]
    f1 = feature[:, DH:]
    r0, r1 = _edge_linear(review_feat, W.T)
    out0, out1 = _sc_call(r0, r1, f0, f1, src, dst, ci_flat)
    return jnp.concatenate([out0, out1], axis=1)


# speedup vs baseline: 5.1677x; 1.3565x over previous
import functools

import jax
import jax.numpy as jnp
from jax import lax
from jax.experimental import pallas as pl
from jax.experimental.pallas import tpu as pltpu
from jax.experimental.pallas import tpu_sc as plsc

N_NODES = 50000
N_EDGES = 800000
D = 64
DH = 32
L = 16
NC, NS = 2, 16
CHUNK = 128
N_CHUNKS = N_EDGES // CHUNK
ROW_CHUNK = 128
N_ROW_CHUNKS = N_NODES // ROW_CHUNK
ROW_TAIL = N_NODES - N_ROW_CHUNKS * ROW_CHUNK
MM_BLK = 4000


def _matmul_body(rf_ref, wt_ref, r0_ref, r1_ref):
    y = jnp.dot(rf_ref[...], wt_ref[...], preferred_element_type=jnp.float32)
    r0_ref[...] = y[:, :DH]
    r1_ref[...] = y[:, DH:]


def _edge_linear(review_feat, w_t):
    return pl.pallas_call(
        _matmul_body,
        grid=(N_EDGES // MM_BLK,),
        in_specs=[
            pl.BlockSpec((MM_BLK, D), lambda i: (i, 0)),
            pl.BlockSpec((D, D), lambda i: (0, 0)),
        ],
        out_specs=[
            pl.BlockSpec((MM_BLK, DH), lambda i: (i, 0)),
            pl.BlockSpec((MM_BLK, DH), lambda i: (i, 0)),
        ],
        out_shape=[
            jax.ShapeDtypeStruct((N_EDGES, DH), jnp.float32),
            jax.ShapeDtypeStruct((N_EDGES, DH), jnp.float32),
        ],
    )(review_feat, w_t)


def _feat_split_body(f_ref, f0_ref, f1_ref):
    f = f_ref[...]
    f0_ref[...] = f[:, :DH]
    f1_ref[...] = f[:, DH:]


def _feat_split(feature):
    blk = 2000
    return pl.pallas_call(
        _feat_split_body,
        grid=(N_NODES // blk,),
        in_specs=[pl.BlockSpec((blk, D), lambda i: (i, 0))],
        out_specs=[
            pl.BlockSpec((blk, DH), lambda i: (i, 0)),
            pl.BlockSpec((blk, DH), lambda i: (i, 0)),
        ],
        out_shape=[
            jax.ShapeDtypeStruct((N_NODES, DH), jnp.float32),
            jax.ShapeDtypeStruct((N_NODES, DH), jnp.float32),
        ],
    )(feature)


def _sc_body(r_hbm, f_hbm, src_hbm, dst_hbm, ci_hbm, out_hbm, col,
             h_sh, cig_v, cin_v, src_v, dst_v, r_v, f_v, m_v, d_v,
             semA, semB0, semB1, s):
    def _zero_row(i, _):
        zero = jnp.zeros((L,), jnp.float32)
        d_v[i, pl.ds(0, L)] = zero
        d_v[i, pl.ds(L, L)] = zero
        return 0

    lax.fori_loop(0, ROW_CHUNK, _zero_row, 0)

    n_rc = N_ROW_CHUNKS // NS + jnp.where(s < N_ROW_CHUNKS % NS, 1, 0)

    def _zero_chunk(k, _):
        base = (s + NS * k) * ROW_CHUNK
        pltpu.sync_copy(d_v, h_sh.at[pl.ds(base, ROW_CHUNK)])
        return 0

    lax.fori_loop(0, n_rc, _zero_chunk, 0)

    @pl.when(s == NS - 1)
    def _():
        pltpu.sync_copy(d_v.at[pl.ds(0, ROW_TAIL)],
                        h_sh.at[pl.ds(N_ROW_CHUNKS * ROW_CHUNK, ROW_TAIL)])

    plsc.subcore_barrier()

    n_my = N_CHUNKS // NS + jnp.where(s < N_CHUNKS % NS, 1, 0)
    semB = (semB0, semB1)

    def _sl(k, b):
        @pl.when(k < n_my)
        def _():
            e0 = (s + NS * k) * CHUNK
            pltpu.async_copy(src_hbm.at[pl.ds(e0, CHUNK)], src_v.at[b], semA)
            pltpu.async_copy(dst_hbm.at[pl.ds(e0, CHUNK)], dst_v.at[b], semA)
            pltpu.async_copy(r_hbm.at[pl.ds(e0, CHUNK)], r_v.at[b], semA)

    def _wl(k, b):
        @pl.when(k < n_my)
        def _():
            pltpu.make_async_copy(src_hbm.at[pl.ds(0, CHUNK)], src_v.at[b], semA).wait()
            pltpu.make_async_copy(dst_hbm.at[pl.ds(0, CHUNK)], dst_v.at[b], semA).wait()
            pltpu.make_async_copy(r_hbm.at[pl.ds(0, CHUNK)], r_v.at[b], semA).wait()

    def _sg(k, b):
        @pl.when(k < n_my)
        def _():
            pltpu.async_copy(f_hbm.at[src_v.at[b]], f_v.at[b], semB[b])
            pltpu.async_copy(ci_hbm.at[src_v.at[b]], cig_v.at[b], semB[b])

    def _wg(k, b):
        @pl.when(k < n_my)
        def _():
            pltpu.make_async_copy(f_hbm.at[src_v.at[b]], f_v.at[b], semB[b]).wait()
            pltpu.make_async_copy(ci_hbm.at[src_v.at[b]], cig_v.at[b], semB[b]).wait()

    def _cs(k, b):
        @pl.when(k < n_my)
        def _():
            def _group(g, _):
                base = g * L
                civ = cig_v[b, pl.ds(base, L)]
                for j in range(L):
                    e = base + j
                    cse = civ[j]
                    m_v[e, pl.ds(0, L)] = (
                        r_v[b, e, pl.ds(0, L)] + f_v[b, e, pl.ds(0, L)]) * cse
                    m_v[e, pl.ds(L, L)] = (
                        r_v[b, e, pl.ds(L, L)] + f_v[b, e, pl.ds(L, L)]) * cse
                return 0

            lax.fori_loop(0, CHUNK // L, _group, 0)
            pltpu.sync_copy(m_v, h_sh.at[dst_v.at[b]], add=True)

    def _iter(k, b):
        _wl(k + 1, 1 - b)
        _sg(k + 1, 1 - b)
        _wg(k, b)
        _cs(k, b)
        _sl(k + 2, b)

    _sl(0, 0)
    _wl(0, 0)
    _sg(0, 0)
    _sl(1, 1)

    n_pairs = (N_CHUNKS // NS + 1) // 2

    def _pair(p, _):
        k = 2 * p
        _iter(k, 0)
        _iter(k + 1, 1)
        return 0

    lax.fori_loop(0, n_pairs, _pair, 0)
    _iter(2 * n_pairs, 0)
    plsc.subcore_barrier()

    def _scale_rows(ngroups):
        def _rgroup(g, _):
            rb = g * L
            civ = cin_v[pl.ds(rb, L)]
            for j in range(L):
                cn = civ[j]
                d_v[rb + j, pl.ds(0, L)] = d_v[rb + j, pl.ds(0, L)] * cn
                d_v[rb + j, pl.ds(L, L)] = d_v[rb + j, pl.ds(L, L)] * cn
            return 0

        lax.fori_loop(0, ngroups, _rgroup, 0)

    def _drain_chunk(k, _):
        base = (s + NS * k) * ROW_CHUNK
        pltpu.sync_copy(h_sh.at[pl.ds(base, ROW_CHUNK)], d_v)
        pltpu.sync_copy(ci_hbm.at[pl.ds(base, ROW_CHUNK)], cin_v)
        _scale_rows(ROW_CHUNK // L)
        pltpu.sync_copy(d_v, out_hbm.at[pl.ds(base, ROW_CHUNK), pl.ds(col, DH)])
        return 0

    lax.fori_loop(0, n_rc, _drain_chunk, 0)

    @pl.when(s == NS - 1)
    def _():
        base = N_ROW_CHUNKS * ROW_CHUNK
        pltpu.sync_copy(h_sh.at[pl.ds(base, ROW_TAIL)], d_v.at[pl.ds(0, ROW_TAIL)])
        pltpu.sync_copy(ci_hbm.at[pl.ds(base, ROW_TAIL)], cin_v.at[pl.ds(0, ROW_TAIL)])
        _scale_rows(ROW_TAIL // L)
        pltpu.sync_copy(d_v.at[pl.ds(0, ROW_TAIL)],
                        out_hbm.at[pl.ds(base, ROW_TAIL), pl.ds(col, DH)])


def _sc_kernel(r0, r1, f0, f1, src_hbm, dst_hbm, ci_hbm, out_hbm,
               h_sh, cig_v, cin_v, src_v, dst_v, r_v, f_v, m_v, d_v,
               semA, semB0, semB1):
    c = lax.axis_index("c")
    s = lax.axis_index("s")

    @pl.when(c == 0)
    def _():
        _sc_body(r0, f0, src_hbm, dst_hbm, ci_hbm, out_hbm, 0,
                 h_sh, cig_v, cin_v, src_v, dst_v, r_v, f_v, m_v, d_v,
                 semA, semB0, semB1, s)

    @pl.when(c == 1)
    def _():
        _sc_body(r1, f1, src_hbm, dst_hbm, ci_hbm, out_hbm, DH,
                 h_sh, cig_v, cin_v, src_v, dst_v, r_v, f_v, m_v, d_v,
                 semA, semB0, semB1, s)


_sc_call = functools.partial(
    pl.kernel,
    _sc_kernel,
    out_type=jax.ShapeDtypeStruct((N_NODES, D), jnp.float32),
    mesh=plsc.VectorSubcoreMesh(
        core_axis_name="c", subcore_axis_name="s",
        num_cores=NC, num_subcores=NS),
    scratch_types=[
        pltpu.VMEM_SHARED((N_NODES, DH), jnp.float32),
        pltpu.VMEM((2, CHUNK), jnp.float32),
        pltpu.VMEM((ROW_CHUNK,), jnp.float32),
        pltpu.VMEM((2, CHUNK), jnp.int32),
        pltpu.VMEM((2, CHUNK), jnp.int32),
        pltpu.VMEM((2, CHUNK, DH), jnp.float32),
        pltpu.VMEM((2, CHUNK, DH), jnp.float32),
        pltpu.VMEM((CHUNK, DH), jnp.float32),
        pltpu.VMEM((ROW_CHUNK, DH), jnp.float32),
        pltpu.SemaphoreType.DMA,
        pltpu.SemaphoreType.DMA,
        pltpu.SemaphoreType.DMA,
    ],
    compiler_params=pltpu.CompilerParams(use_tc_tiling_on_sc=False),
)()


@jax.jit
def kernel(feature, edge_index, review_feat, ci, W):
    src = edge_index[0].astype(jnp.int32)
    dst = edge_index[1].astype(jnp.int32)
    ci_flat = ci[:, 0]
    f0, f1 = _feat_split(feature)
    r0, r1 = _edge_linear(review_feat, W.T)
    return _sc_call(r0, r1, f0, f1, src, dst, ci_flat)
